# baseline (speedup 1.0000x reference)
import functools

import jax
import jax.numpy as jnp
from jax import lax
from jax.experimental import pallas as pl
from jax.experimental.pallas import tpu as pltpu
from jax.experimental.pallas import tpu_sc as plsc

N = 50000
NP = 51200
E = 200000
EP = 212992
CHUNKS = 104
GROUP = 8
AGRP = 4
STRIPE = 3200
TROW = 3200
W = 16
DUMMY = N

_mesh1 = plsc.VectorSubcoreMesh(
    core_axis_name="c", subcore_axis_name="s", num_cores=1)


@functools.partial(
    pl.kernel,
    out_type=[jax.ShapeDtypeStruct((NP,), jnp.float32)] * 3,
    mesh=_mesh1,
    scratch_types=[
        pltpu.VMEM_SHARED((NP,), jnp.float32),
        pltpu.VMEM((CHUNKS, 128), jnp.int32),
        pltpu.VMEM((128,), jnp.float32),
        pltpu.VMEM((STRIPE,), jnp.float32),
        pltpu.SemaphoreType.DMA,
    ],
)
def _deg_kernel(ones_h, zer_h, e0, e1, e2, o0, o1, o2,
                acc, idxv, ones_v, zvm, sem):
  sid = lax.axis_index("s")
  pltpu.sync_copy(ones_h, ones_v)
  pltpu.sync_copy(zer_h, zvm)

  def task(eidx, out):
    pltpu.sync_copy(zvm, acc.at[pl.ds(sid * STRIPE, STRIPE)])
    plsc.subcore_barrier()
    pltpu.sync_copy(eidx.at[sid], idxv)

    def body(g, carry):
      descs = [
          pltpu.async_copy(ones_v, acc.at[idxv.at[g * GROUP + k]], sem,
                           add=True)
          for k in range(GROUP)
      ]
      for d in descs:
        d.wait()
      return carry
    lax.fori_loop(0, CHUNKS // GROUP, body, 0)
    plsc.subcore_barrier()
    pltpu.sync_copy(acc.at[pl.ds(sid * STRIPE, STRIPE)],
                    out.at[pl.ds(sid * STRIPE, STRIPE)])
    plsc.subcore_barrier()

  task(e0, o0)
  task(e1, o1)
  task(e2, o2)


def _make_agg(ncols):
  @functools.partial(
      pl.kernel,
      out_type=[jax.ShapeDtypeStruct((NP, 128), jnp.float32)] * 3,
      mesh=_mesh1,
      scratch_types=[
          pltpu.VMEM_SHARED((NP, W), jnp.float32),
          pltpu.VMEM((CHUNKS, 128), jnp.int32),
          pltpu.VMEM((CHUNKS, 128), jnp.int32),
          pltpu.VMEM((2 * AGRP, 128, W), jnp.float32),
          pltpu.VMEM((128, W), jnp.float32),
          pltpu.SemaphoreType.DMA,
          pltpu.SemaphoreType.DMA,
      ],
      compiler_params=pltpu.CompilerParams(use_tc_tiling_on_sc=False),
  )
  def agg_kernel(zer_h, f0, f1, f2, s0, s1, s2, d0, d1, d2,
                 o0, o1, o2, acc, idxs, idxd, rows, zvm, gsem, ssem):
    feats = (f0, f1, f2)
    srcs = (s0, s1, s2)
    dsts = (d0, d1, d2)
    outs = (o0, o1, o2)
    sid = lax.axis_index("s")
    pltpu.sync_copy(zer_h, zvm)

    def task(feat, src, dst, out):
      def percol(c, carry):
        zd = [
            pltpu.async_copy(zvm,
                             acc.at[pl.ds(sid * STRIPE + i * 128, 128)],
                             ssem)
            for i in range(STRIPE // 128)
        ]
        for d in zd:
          d.wait()
        plsc.subcore_barrier()
        pltpu.sync_copy(src.at[sid], idxs)
        pltpu.sync_copy(dst.at[sid], idxd)

        def xform(j, xcarry):
          for k in range(8):
            sl = (j, pl.ds(k * 16, 16))
            idxs[sl] = idxs[sl] * 8 + c
          return xcarry
        lax.fori_loop(0, CHUNKS, xform, 0)

        def body(g, bcarry):
          base = g * 2 * AGRP
          gda = [
              pltpu.async_copy(feat.at[idxs.at[base + k]], rows.at[k], gsem)
              for k in range(AGRP)
          ]
          for d in gda:
            d.wait()
          sda = [
              pltpu.async_copy(rows.at[k], acc.at[idxd.at[base + k]],
                               ssem, add=True)
              for k in range(AGRP)
          ]
          gdb = [
              pltpu.async_copy(feat.at[idxs.at[base + AGRP + k]],
                               rows.at[AGRP + k], gsem)
              for k in range(AGRP)
          ]
          for d in gdb:
            d.wait()
          for d in sda:
            d.wait()
          sdb = [
              pltpu.async_copy(rows.at[AGRP + k],
                               acc.at[idxd.at[base + AGRP + k]],
                               ssem, add=True)
              for k in range(AGRP)
          ]
          for d in sdb:
            d.wait()
          return bcarry
        lax.fori_loop(0, CHUNKS // (2 * AGRP), body, 0)
        plsc.subcore_barrier()
        pltpu.sync_copy(acc.at[pl.ds(sid * STRIPE, STRIPE)],
                        out.at[pl.ds(sid * STRIPE, STRIPE),
                               pl.ds(c * W, W)])
        plsc.subcore_barrier()
        return carry
      lax.fori_loop(0, ncols // W, percol, 0)

    for r in range(3):
      task(feats[r], srcs[r], dsts[r], outs[r])

  return agg_kernel


_agg_l1 = _make_agg(128)
_agg_l2 = _make_agg(48)


def _scale_body(d0, d1, d2, x_ref, f0, f1, f2):
  xv = x_ref[...]
  for dref, fref in ((d0, f0), (d1, f1), (d2, f2)):
    s = lax.rsqrt(jnp.maximum(dref[...], 1.0))
    fref[...] = xv * s


def _l1_body(i0, i1, i2, b10, b11, b12, w10, w11, w12, a0, a1, a2, h_ref):
  acc = jnp.zeros((TROW, 128), jnp.float32)
  for iref, wref, aref in ((i0, w10, a0), (i1, w11, a1), (i2, w12, a2)):
    s = lax.rsqrt(jnp.maximum(iref[...], 1.0))
    acc += jnp.dot(aref[...] * s, wref[...],
                   preferred_element_type=jnp.float32)
  bbar = (b10[...] + b11[...] + b12[...]) * (1.0 / 3.0)
  h_ref[...] = jnp.maximum(acc * (1.0 / 3.0) + bbar, 0.0)


def _l2pre_body(o0, o1, o2, wc_ref, w20, w21, w22, h_ref, q0, q1, q2):
  hv = h_ref[...]
  wcp = jnp.concatenate(
      [wc_ref[...], jnp.zeros((128, 88), jnp.float32)], axis=1)
  for oref, wref, qref in ((o0, w20, q0), (o1, w21, q1), (o2, w22, q2)):
    m = jnp.dot(wref[...], wcp, preferred_element_type=jnp.float32)
    s = lax.rsqrt(jnp.maximum(oref[...], 1.0))
    qref[...] = jnp.dot(hv * s, m, preferred_element_type=jnp.float32)


def _final_body(i0, i1, i2, b20, b21, b22, wc_ref, bc_ref, a0, a1, a2, out):
  acc = jnp.zeros((TROW, 48), jnp.float32)
  for iref, aref in ((i0, a0), (i1, a1), (i2, a2)):
    s = lax.rsqrt(jnp.maximum(iref[...], 1.0))
    acc += aref[...][:, :48] * s
  bb = jnp.dot((b20[...] + b21[...] + b22[...]) * (1.0 / 3.0), wc_ref[...],
               preferred_element_type=jnp.float32)
  out[...] = acc[:, :40] * (1.0 / 3.0) + bb + bc_ref[...]


def _row_spec(w):
  return pl.BlockSpec((TROW, w), lambda i: (i, 0))


def _full_spec(a, b):
  return pl.BlockSpec((a, b), lambda i: (0, 0))


def kernel(x, edge_index_r0, edge_index_r1, edge_index_r2,
           W1_0, b1_0, W1_1, b1_1, W1_2, b1_2,
           W2_0, b2_0, W2_1, b2_1, W2_2, b2_2,
           Wc, bc):
  xp = jnp.pad(x, ((0, NP - N), (0, 0)))

  pad = DUMMY + (jnp.arange(EP - E, dtype=jnp.int32) % (NP - N - 8))

  def prep(ei):
    s = jnp.concatenate([ei[0].astype(jnp.int32), pad]).reshape(16, CHUNKS, 128)
    d = jnp.concatenate([ei[1].astype(jnp.int32), pad]).reshape(16, CHUNKS, 128)
    return s, d

  s0, d0 = prep(edge_index_r0)
  s1, d1 = prep(edge_index_r1)
  s2, d2 = prep(edge_index_r2)

  ones128 = jnp.ones((128,), jnp.float32)
  zer1d = jnp.zeros((STRIPE,), jnp.float32)
  zer2d = jnp.zeros((128, W), jnp.float32)

  od0, od1, od2 = _deg_kernel(ones128, zer1d, s0, s1, s2)
  id0, id1, id2 = _deg_kernel(ones128, zer1d, d0, d1, d2)
  od = [d.reshape(NP, 1) for d in (od0, od1, od2)]
  idg = [d.reshape(NP, 1) for d in (id0, id1, id2)]

  feats = pl.pallas_call(
      _scale_body,
      grid=(NP // TROW,),
      in_specs=[_row_spec(1)] * 3 + [_row_spec(128)],
      out_specs=[_row_spec(128)] * 3,
      out_shape=[jax.ShapeDtypeStruct((NP, 128), jnp.float32)] * 3,
  )(od[0], od[1], od[2], xp)

  fviews = [f.reshape(8 * NP, W) for f in feats]
  aggs1 = _agg_l1(zer2d, *fviews, s0, s1, s2, d0, d1, d2)

  b1 = [b.reshape(1, 128) for b in (b1_0, b1_1, b1_2)]
  h = pl.pallas_call(
      _l1_body,
      grid=(NP // TROW,),
      in_specs=([_row_spec(1)] * 3 + [_full_spec(1, 128)] * 3
                + [_full_spec(128, 128)] * 3 + [_row_spec(128)] * 3),
      out_specs=_row_spec(128),
      out_shape=jax.ShapeDtypeStruct((NP, 128), jnp.float32),
  )(idg[0], idg[1], idg[2], *b1, W1_0, W1_1, W1_2, *aggs1)

  qs = pl.pallas_call(
      _l2pre_body,
      grid=(NP // TROW,),
      in_specs=([_row_spec(1)] * 3 + [_full_spec(128, 40)]
                + [_full_spec(128, 128)] * 3 + [_row_spec(128)]),
      out_specs=[_row_spec(128)] * 3,
      out_shape=[jax.ShapeDtypeStruct((NP, 128), jnp.float32)] * 3,
  )(od[0], od[1], od[2], Wc, W2_0, W2_1, W2_2, h)

  qviews = [q.reshape(8 * NP, W) for q in qs]
  aggs2 = _agg_l2(zer2d, *qviews, s0, s1, s2, d0, d1, d2)

  b2 = [b.reshape(1, 128) for b in (b2_0, b2_1, b2_2)]
  logits = pl.pallas_call(
      _final_body,
      grid=(NP // TROW,),
      in_specs=([_row_spec(1)] * 3 + [_full_spec(1, 128)] * 3
                + [_full_spec(128, 40)] + [_full_spec(1, 40)]
                + [_row_spec(128)] * 3),
      out_specs=_row_spec(40),
      out_shape=jax.ShapeDtypeStruct((NP, 40), jnp.float32),
  )(idg[0], idg[1], idg[2], *b2, Wc, bc.reshape(1, 40), *aggs2)

  return logits[:N]


# speedup vs baseline: 1.1279x; 1.1279x over previous
import functools

import jax
import jax.numpy as jnp
from jax import lax
from jax.experimental import pallas as pl
from jax.experimental.pallas import tpu as pltpu
from jax.experimental.pallas import tpu_sc as plsc

N = 50000
NP = 51200
E = 200000
EP = 212992
CHUNKS = 104
GROUP = 8
AGRP = 13
STRIPE = 3200
TROW = 3200
W = 16
DUMMY = N

_mesh1 = plsc.VectorSubcoreMesh(
    core_axis_name="c", subcore_axis_name="s", num_cores=1)


@functools.partial(
    pl.kernel,
    out_type=[jax.ShapeDtypeStruct((NP,), jnp.float32)] * 3,
    mesh=_mesh1,
    scratch_types=[
        pltpu.VMEM_SHARED((NP,), jnp.float32),
        pltpu.VMEM((CHUNKS, 128), jnp.int32),
        pltpu.VMEM((128,), jnp.float32),
        pltpu.VMEM((STRIPE,), jnp.float32),
        pltpu.SemaphoreType.DMA,
    ],
)
def _deg_kernel(ones_h, zer_h, e0, e1, e2, o0, o1, o2,
                acc, idxv, ones_v, zvm, sem):
  sid = lax.axis_index("s")
  pltpu.sync_copy(ones_h, ones_v)
  pltpu.sync_copy(zer_h, zvm)

  def task(eidx, out):
    pltpu.sync_copy(zvm, acc.at[pl.ds(sid * STRIPE, STRIPE)])
    plsc.subcore_barrier()
    pltpu.sync_copy(eidx.at[sid], idxv)

    def body(g, carry):
      descs = [
          pltpu.async_copy(ones_v, acc.at[idxv.at[g * GROUP + k]], sem,
                           add=True)
          for k in range(GROUP)
      ]
      for d in descs:
        d.wait()
      return carry
    lax.fori_loop(0, CHUNKS // GROUP, body, 0)
    plsc.subcore_barrier()
    pltpu.sync_copy(acc.at[pl.ds(sid * STRIPE, STRIPE)],
                    out.at[pl.ds(sid * STRIPE, STRIPE)])
    plsc.subcore_barrier()

  task(e0, o0)
  task(e1, o1)
  task(e2, o2)


def _make_agg(ncols):
  @functools.partial(
      pl.kernel,
      out_type=[jax.ShapeDtypeStruct((NP, 128), jnp.float32)] * 3,
      mesh=_mesh1,
      scratch_types=[
          pltpu.VMEM_SHARED((NP, W), jnp.float32),
          pltpu.VMEM((CHUNKS, 128), jnp.int32),
          pltpu.VMEM((CHUNKS, 128), jnp.int32),
          pltpu.VMEM((AGRP, 128, W), jnp.float32),
          pltpu.VMEM((128, W), jnp.float32),
          pltpu.SemaphoreType.DMA,
          pltpu.SemaphoreType.DMA,
      ],
      compiler_params=pltpu.CompilerParams(use_tc_tiling_on_sc=False),
  )
  def agg_kernel(zer_h, f0, f1, f2, s0, s1, s2, d0, d1, d2,
                 o0, o1, o2, acc, idxs, idxd, rows, zvm, gsem, ssem):
    feats = (f0, f1, f2)
    srcs = (s0, s1, s2)
    dsts = (d0, d1, d2)
    outs = (o0, o1, o2)
    sid = lax.axis_index("s")
    pltpu.sync_copy(zer_h, zvm)

    def task(feat, src, dst, out):
      def percol(c, carry):
        zd = [
            pltpu.async_copy(zvm,
                             acc.at[pl.ds(sid * STRIPE + i * 128, 128)],
                             ssem)
            for i in range(STRIPE // 128)
        ]
        for d in zd:
          d.wait()
        plsc.subcore_barrier()
        pltpu.sync_copy(src.at[sid], idxs)
        pltpu.sync_copy(dst.at[sid], idxd)

        def xform(j, xcarry):
          for k in range(8):
            sl = (j, pl.ds(k * 16, 16))
            idxs[sl] = idxs[sl] * 8 + c
          return xcarry
        lax.fori_loop(0, CHUNKS, xform, 0)

        def body(g, bcarry):
          base = g * AGRP
          gd = [
              pltpu.async_copy(feat.at[idxs.at[base + k]], rows.at[k], gsem)
              for k in range(AGRP)
          ]
          for d in gd:
            d.wait()
          sd = [
              pltpu.async_copy(rows.at[k], acc.at[idxd.at[base + k]],
                               ssem, add=True)
              for k in range(AGRP)
          ]
          for d in sd:
            d.wait()
          return bcarry
        lax.fori_loop(0, CHUNKS // AGRP, body, 0)
        plsc.subcore_barrier()
        pltpu.sync_copy(acc.at[pl.ds(sid * STRIPE, STRIPE)],
                        out.at[pl.ds(sid * STRIPE, STRIPE),
                               pl.ds(c * W, W)])
        plsc.subcore_barrier()
        return carry
      lax.fori_loop(0, ncols // W, percol, 0)

    for r in range(3):
      task(feats[r], srcs[r], dsts[r], outs[r])

  return agg_kernel


_agg_l1 = _make_agg(128)
_agg_l2 = _make_agg(48)


def _scale_body(d0, d1, d2, x_ref, f0, f1, f2):
  xv = x_ref[...]
  for dref, fref in ((d0, f0), (d1, f1), (d2, f2)):
    s = lax.rsqrt(jnp.maximum(dref[...], 1.0))
    fref[...] = xv * s


def _l1_body(i0, i1, i2, b10, b11, b12, w10, w11, w12, a0, a1, a2, h_ref):
  acc = jnp.zeros((TROW, 128), jnp.float32)
  for iref, wref, aref in ((i0, w10, a0), (i1, w11, a1), (i2, w12, a2)):
    s = lax.rsqrt(jnp.maximum(iref[...], 1.0))
    acc += jnp.dot(aref[...] * s, wref[...],
                   preferred_element_type=jnp.float32)
  bbar = (b10[...] + b11[...] + b12[...]) * (1.0 / 3.0)
  h_ref[...] = jnp.maximum(acc * (1.0 / 3.0) + bbar, 0.0)


def _l2pre_body(o0, o1, o2, wc_ref, w20, w21, w22, h_ref, q0, q1, q2):
  hv = h_ref[...]
  wcp = jnp.concatenate(
      [wc_ref[...], jnp.zeros((128, 88), jnp.float32)], axis=1)
  for oref, wref, qref in ((o0, w20, q0), (o1, w21, q1), (o2, w22, q2)):
    m = jnp.dot(wref[...], wcp, preferred_element_type=jnp.float32)
    s = lax.rsqrt(jnp.maximum(oref[...], 1.0))
    qref[...] = jnp.dot(hv * s, m, preferred_element_type=jnp.float32)


def _final_body(i0, i1, i2, b20, b21, b22, wc_ref, bc_ref, a0, a1, a2, out):
  acc = jnp.zeros((TROW, 48), jnp.float32)
  for iref, aref in ((i0, a0), (i1, a1), (i2, a2)):
    s = lax.rsqrt(jnp.maximum(iref[...], 1.0))
    acc += aref[...][:, :48] * s
  bb = jnp.dot((b20[...] + b21[...] + b22[...]) * (1.0 / 3.0), wc_ref[...],
               preferred_element_type=jnp.float32)
  out[...] = acc[:, :40] * (1.0 / 3.0) + bb + bc_ref[...]


def _row_spec(w):
  return pl.BlockSpec((TROW, w), lambda i: (i, 0))


def _full_spec(a, b):
  return pl.BlockSpec((a, b), lambda i: (0, 0))


def kernel(x, edge_index_r0, edge_index_r1, edge_index_r2,
           W1_0, b1_0, W1_1, b1_1, W1_2, b1_2,
           W2_0, b2_0, W2_1, b2_1, W2_2, b2_2,
           Wc, bc):
  xp = jnp.pad(x, ((0, NP - N), (0, 0)))

  pad = DUMMY + (jnp.arange(EP - E, dtype=jnp.int32) % (NP - N - 8))

  def prep(ei):
    s = jnp.concatenate([ei[0].astype(jnp.int32), pad]).reshape(16, CHUNKS, 128)
    d = jnp.concatenate([ei[1].astype(jnp.int32), pad]).reshape(16, CHUNKS, 128)
    return s, d

  s0, d0 = prep(edge_index_r0)
  s1, d1 = prep(edge_index_r1)
  s2, d2 = prep(edge_index_r2)

  ones128 = jnp.ones((128,), jnp.float32)
  zer1d = jnp.zeros((STRIPE,), jnp.float32)
  zer2d = jnp.zeros((128, W), jnp.float32)

  od0, od1, od2 = _deg_kernel(ones128, zer1d, s0, s1, s2)
  id0, id1, id2 = _deg_kernel(ones128, zer1d, d0, d1, d2)
  od = [d.reshape(NP, 1) for d in (od0, od1, od2)]
  idg = [d.reshape(NP, 1) for d in (id0, id1, id2)]

  feats = pl.pallas_call(
      _scale_body,
      grid=(NP // TROW,),
      in_specs=[_row_spec(1)] * 3 + [_row_spec(128)],
      out_specs=[_row_spec(128)] * 3,
      out_shape=[jax.ShapeDtypeStruct((NP, 128), jnp.float32)] * 3,
  )(od[0], od[1], od[2], xp)

  fviews = [f.reshape(8 * NP, W) for f in feats]
  aggs1 = _agg_l1(zer2d, *fviews, s0, s1, s2, d0, d1, d2)

  b1 = [b.reshape(1, 128) for b in (b1_0, b1_1, b1_2)]
  h = pl.pallas_call(
      _l1_body,
      grid=(NP // TROW,),
      in_specs=([_row_spec(1)] * 3 + [_full_spec(1, 128)] * 3
                + [_full_spec(128, 128)] * 3 + [_row_spec(128)] * 3),
      out_specs=_row_spec(128),
      out_shape=jax.ShapeDtypeStruct((NP, 128), jnp.float32),
  )(idg[0], idg[1], idg[2], *b1, W1_0, W1_1, W1_2, *aggs1)

  qs = pl.pallas_call(
      _l2pre_body,
      grid=(NP // TROW,),
      in_specs=([_row_spec(1)] * 3 + [_full_spec(128, 40)]
                + [_full_spec(128, 128)] * 3 + [_row_spec(128)]),
      out_specs=[_row_spec(128)] * 3,
      out_shape=[jax.ShapeDtypeStruct((NP, 128), jnp.float32)] * 3,
  )(od[0], od[1], od[2], Wc, W2_0, W2_1, W2_2, h)

  qviews = [q.reshape(8 * NP, W) for q in qs]
  aggs2 = _agg_l2(zer2d, *qviews, s0, s1, s2, d0, d1, d2)

  b2 = [b.reshape(1, 128) for b in (b2_0, b2_1, b2_2)]
  logits = pl.pallas_call(
      _final_body,
      grid=(NP // TROW,),
      in_specs=([_row_spec(1)] * 3 + [_full_spec(1, 128)] * 3
                + [_full_spec(128, 40)] + [_full_spec(1, 40)]
                + [_row_spec(128)] * 3),
      out_specs=_row_spec(40),
      out_shape=jax.ShapeDtypeStruct((NP, 40), jnp.float32),
  )(idg[0], idg[1], idg[2], *b2, Wc, bc.reshape(1, 40), *aggs2)

  return logits[:N]


# speedup vs baseline: 1.1308x; 1.0026x over previous
import functools

import jax
import jax.numpy as jnp
from jax import lax
from jax.experimental import pallas as pl
from jax.experimental.pallas import tpu as pltpu
from jax.experimental.pallas import tpu_sc as plsc

N = 50000
NP = 51200
E = 200000
EP = 212992
CHUNKS = 104
GROUP = 8
AGRP = 13
STRIPE = 3200
TROW = 6400
W = 16
DUMMY = N

_mesh1 = plsc.VectorSubcoreMesh(
    core_axis_name="c", subcore_axis_name="s", num_cores=1)


@functools.partial(
    pl.kernel,
    out_type=[jax.ShapeDtypeStruct((NP,), jnp.float32)] * 6,
    mesh=_mesh1,
    scratch_types=[
        pltpu.VMEM_SHARED((NP,), jnp.float32),
        pltpu.VMEM((CHUNKS, 128), jnp.int32),
        pltpu.VMEM((128,), jnp.float32),
        pltpu.VMEM((STRIPE,), jnp.float32),
        pltpu.SemaphoreType.DMA,
    ],
)
def _deg_kernel(ones_h, zer_h, e0, e1, e2, e3, e4, e5,
                o0, o1, o2, o3, o4, o5, acc, idxv, ones_v, zvm, sem):
  sid = lax.axis_index("s")
  pltpu.sync_copy(ones_h, ones_v)
  pltpu.sync_copy(zer_h, zvm)

  def task(eidx, out):
    pltpu.sync_copy(zvm, acc.at[pl.ds(sid * STRIPE, STRIPE)])
    plsc.subcore_barrier()
    pltpu.sync_copy(eidx.at[sid], idxv)

    def body(g, carry):
      descs = [
          pltpu.async_copy(ones_v, acc.at[idxv.at[g * GROUP + k]], sem,
                           add=True)
          for k in range(GROUP)
      ]
      for d in descs:
        d.wait()
      return carry
    lax.fori_loop(0, CHUNKS // GROUP, body, 0)
    plsc.subcore_barrier()
    pltpu.sync_copy(acc.at[pl.ds(sid * STRIPE, STRIPE)],
                    out.at[pl.ds(sid * STRIPE, STRIPE)])
    plsc.subcore_barrier()

  task(e0, o0)
  task(e1, o1)
  task(e2, o2)
  task(e3, o3)
  task(e4, o4)
  task(e5, o5)


def _make_agg(ncols):
  @functools.partial(
      pl.kernel,
      out_type=[jax.ShapeDtypeStruct((NP, 128), jnp.float32)] * 3,
      mesh=_mesh1,
      scratch_types=[
          pltpu.VMEM_SHARED((NP, W), jnp.float32),
          pltpu.VMEM((CHUNKS, 128), jnp.int32),
          pltpu.VMEM((CHUNKS, 128), jnp.int32),
          pltpu.VMEM((AGRP, 128, W), jnp.float32),
          pltpu.VMEM((128, W), jnp.float32),
          pltpu.SemaphoreType.DMA,
          pltpu.SemaphoreType.DMA,
      ],
      compiler_params=pltpu.CompilerParams(use_tc_tiling_on_sc=False),
  )
  def agg_kernel(zer_h, f0, f1, f2, s0, s1, s2, d0, d1, d2,
                 o0, o1, o2, acc, idxs, idxd, rows, zvm, gsem, ssem):
    feats = (f0, f1, f2)
    srcs = (s0, s1, s2)
    dsts = (d0, d1, d2)
    outs = (o0, o1, o2)
    sid = lax.axis_index("s")
    pltpu.sync_copy(zer_h, zvm)

    def task(feat, src, dst, out):
      pltpu.sync_copy(dst.at[sid], idxd)

      def percol(c, carry):
        zd = [
            pltpu.async_copy(zvm,
                             acc.at[pl.ds(sid * STRIPE + i * 128, 128)],
                             ssem)
            for i in range(STRIPE // 128)
        ]
        for d in zd:
          d.wait()
        plsc.subcore_barrier()
        pltpu.sync_copy(src.at[sid], idxs)

        def xform(j, xcarry):
          for k in range(8):
            sl = (j, pl.ds(k * 16, 16))
            idxs[sl] = idxs[sl] * 8 + c
          return xcarry
        lax.fori_loop(0, CHUNKS, xform, 0)

        def body(g, bcarry):
          base = g * AGRP
          gd = [
              pltpu.async_copy(feat.at[idxs.at[base + k]], rows.at[k], gsem)
              for k in range(AGRP)
          ]
          for d in gd:
            d.wait()
          sd = [
              pltpu.async_copy(rows.at[k], acc.at[idxd.at[base + k]],
                               ssem, add=True)
              for k in range(AGRP)
          ]
          for d in sd:
            d.wait()
          return bcarry
        lax.fori_loop(0, CHUNKS // AGRP, body, 0)
        plsc.subcore_barrier()
        pltpu.sync_copy(acc.at[pl.ds(sid * STRIPE, STRIPE)],
                        out.at[pl.ds(sid * STRIPE, STRIPE),
                               pl.ds(c * W, W)])
        plsc.subcore_barrier()
        return carry
      lax.fori_loop(0, ncols // W, percol, 0)

    for r in range(3):
      task(feats[r], srcs[r], dsts[r], outs[r])

  return agg_kernel


_agg_l1 = _make_agg(128)
_agg_l2 = _make_agg(48)


def _scale_body(d0, d1, d2, x_ref, f0, f1, f2):
  xv = x_ref[...]
  for dref, fref in ((d0, f0), (d1, f1), (d2, f2)):
    s = lax.rsqrt(jnp.maximum(dref[...], 1.0))
    fref[...] = xv * s


def _l1_body(i0, i1, i2, b10, b11, b12, w10, w11, w12, a0, a1, a2, h_ref):
  acc = jnp.zeros((TROW, 128), jnp.float32)
  for iref, wref, aref in ((i0, w10, a0), (i1, w11, a1), (i2, w12, a2)):
    s = lax.rsqrt(jnp.maximum(iref[...], 1.0))
    acc += jnp.dot(aref[...] * s, wref[...],
                   preferred_element_type=jnp.float32)
  bbar = (b10[...] + b11[...] + b12[...]) * (1.0 / 3.0)
  h_ref[...] = jnp.maximum(acc * (1.0 / 3.0) + bbar, 0.0)


def _l2pre_body(o0, o1, o2, wc_ref, w20, w21, w22, h_ref, q0, q1, q2):
  hv = h_ref[...]
  wcp = jnp.concatenate(
      [wc_ref[...], jnp.zeros((128, 88), jnp.float32)], axis=1)
  for oref, wref, qref in ((o0, w20, q0), (o1, w21, q1), (o2, w22, q2)):
    m = jnp.dot(wref[...], wcp, preferred_element_type=jnp.float32)
    s = lax.rsqrt(jnp.maximum(oref[...], 1.0))
    qref[...] = jnp.dot(hv * s, m, preferred_element_type=jnp.float32)


def _final_body(i0, i1, i2, b20, b21, b22, wc_ref, bc_ref, a0, a1, a2, out):
  acc = jnp.zeros((TROW, 48), jnp.float32)
  for iref, aref in ((i0, a0), (i1, a1), (i2, a2)):
    s = lax.rsqrt(jnp.maximum(iref[...], 1.0))
    acc += aref[...][:, :48] * s
  bb = jnp.dot((b20[...] + b21[...] + b22[...]) * (1.0 / 3.0), wc_ref[...],
               preferred_element_type=jnp.float32)
  out[...] = acc[:, :40] * (1.0 / 3.0) + bb + bc_ref[...]


def _row_spec(w):
  return pl.BlockSpec((TROW, w), lambda i: (i, 0))


def _full_spec(a, b):
  return pl.BlockSpec((a, b), lambda i: (0, 0))


def kernel(x, edge_index_r0, edge_index_r1, edge_index_r2,
           W1_0, b1_0, W1_1, b1_1, W1_2, b1_2,
           W2_0, b2_0, W2_1, b2_1, W2_2, b2_2,
           Wc, bc):
  xp = jnp.pad(x, ((0, NP - N), (0, 0)))

  pad = DUMMY + (jnp.arange(EP - E, dtype=jnp.int32) % (NP - N - 8))

  def prep(ei):
    s = jnp.concatenate([ei[0].astype(jnp.int32), pad]).reshape(16, CHUNKS, 128)
    d = jnp.concatenate([ei[1].astype(jnp.int32), pad]).reshape(16, CHUNKS, 128)
    return s, d

  s0, d0 = prep(edge_index_r0)
  s1, d1 = prep(edge_index_r1)
  s2, d2 = prep(edge_index_r2)

  ones128 = jnp.ones((128,), jnp.float32)
  zer1d = jnp.zeros((STRIPE,), jnp.float32)
  zer2d = jnp.zeros((128, W), jnp.float32)

  od0, od1, od2, id0, id1, id2 = _deg_kernel(
      ones128, zer1d, s0, s1, s2, d0, d1, d2)
  od = [d.reshape(NP, 1) for d in (od0, od1, od2)]
  idg = [d.reshape(NP, 1) for d in (id0, id1, id2)]

  feats = pl.pallas_call(
      _scale_body,
      grid=(NP // TROW,),
      in_specs=[_row_spec(1)] * 3 + [_row_spec(128)],
      out_specs=[_row_spec(128)] * 3,
      out_shape=[jax.ShapeDtypeStruct((NP, 128), jnp.float32)] * 3,
  )(od[0], od[1], od[2], xp)

  fviews = [f.reshape(8 * NP, W) for f in feats]
  aggs1 = _agg_l1(zer2d, *fviews, s0, s1, s2, d0, d1, d2)

  b1 = [b.reshape(1, 128) for b in (b1_0, b1_1, b1_2)]
  h = pl.pallas_call(
      _l1_body,
      grid=(NP // TROW,),
      in_specs=([_row_spec(1)] * 3 + [_full_spec(1, 128)] * 3
                + [_full_spec(128, 128)] * 3 + [_row_spec(128)] * 3),
      out_specs=_row_spec(128),
      out_shape=jax.ShapeDtypeStruct((NP, 128), jnp.float32),
  )(idg[0], idg[1], idg[2], *b1, W1_0, W1_1, W1_2, *aggs1)

  qs = pl.pallas_call(
      _l2pre_body,
      grid=(NP // TROW,),
      in_specs=([_row_spec(1)] * 3 + [_full_spec(128, 40)]
                + [_full_spec(128, 128)] * 3 + [_row_spec(128)]),
      out_specs=[_row_spec(128)] * 3,
      out_shape=[jax.ShapeDtypeStruct((NP, 128), jnp.float32)] * 3,
  )(od[0], od[1], od[2], Wc, W2_0, W2_1, W2_2, h)

  qviews = [q.reshape(8 * NP, W) for q in qs]
  aggs2 = _agg_l2(zer2d, *qviews, s0, s1, s2, d0, d1, d2)

  b2 = [b.reshape(1, 128) for b in (b2_0, b2_1, b2_2)]
  logits = pl.pallas_call(
      _final_body,
      grid=(NP // TROW,),
      in_specs=([_row_spec(1)] * 3 + [_full_spec(1, 128)] * 3
                + [_full_spec(128, 40)] + [_full_spec(1, 40)]
                + [_row_spec(128)] * 3),
      out_specs=_row_spec(40),
      out_shape=jax.ShapeDtypeStruct((NP, 40), jnp.float32),
  )(idg[0], idg[1], idg[2], *b2, Wc, bc.reshape(1, 40), *aggs2)

  return logits[:N]


# speedup vs baseline: 1.2148x; 1.0743x over previous
import functools

import jax
import jax.numpy as jnp
from jax import lax
from jax.experimental import pallas as pl
from jax.experimental.pallas import tpu as pltpu
from jax.experimental.pallas import tpu_sc as plsc

N = 50000
NP = 51200
E = 200000
EP = 212992
CHUNKS = 104
GROUP = 8
AGRP = 13
STRIPE = 3200
TROW = 6400
W = 16
DUMMY = N

_mesh1 = plsc.VectorSubcoreMesh(
    core_axis_name="c", subcore_axis_name="s", num_cores=1)


@functools.partial(
    pl.kernel,
    out_type=[jax.ShapeDtypeStruct((NP,), jnp.float32)] * 6,
    mesh=_mesh1,
    scratch_types=[
        pltpu.VMEM_SHARED((NP,), jnp.float32),
        pltpu.VMEM((CHUNKS, 128), jnp.int32),
        pltpu.VMEM((128,), jnp.float32),
        pltpu.VMEM((STRIPE,), jnp.float32),
        pltpu.SemaphoreType.DMA,
    ],
)
def _deg_kernel(ones_h, zer_h, e0, e1, e2, e3, e4, e5,
                o0, o1, o2, o3, o4, o5, acc, idxv, ones_v, zvm, sem):
  sid = lax.axis_index("s")
  pltpu.sync_copy(ones_h, ones_v)
  pltpu.sync_copy(zer_h, zvm)

  def task(eidx, out):
    pltpu.sync_copy(zvm, acc.at[pl.ds(sid * STRIPE, STRIPE)])
    plsc.subcore_barrier()
    pltpu.sync_copy(eidx.at[sid], idxv)

    def body(g, carry):
      descs = [
          pltpu.async_copy(ones_v, acc.at[idxv.at[g * GROUP + k]], sem,
                           add=True)
          for k in range(GROUP)
      ]
      for d in descs:
        d.wait()
      return carry
    lax.fori_loop(0, CHUNKS // GROUP, body, 0)
    plsc.subcore_barrier()
    pltpu.sync_copy(acc.at[pl.ds(sid * STRIPE, STRIPE)],
                    out.at[pl.ds(sid * STRIPE, STRIPE)])
    plsc.subcore_barrier()

  task(e0, o0)
  task(e1, o1)
  task(e2, o2)
  task(e3, o3)
  task(e4, o4)
  task(e5, o5)


def _make_agg(ncols):
  @functools.partial(
      pl.kernel,
      out_type=[jax.ShapeDtypeStruct((NP, 128), jnp.float32)] * 3,
      mesh=_mesh1,
      scratch_types=[
          pltpu.VMEM_SHARED((NP, W), jnp.float32),
          pltpu.VMEM((CHUNKS // 2, 128), jnp.int32),
          pltpu.VMEM((CHUNKS // 2, 128), jnp.int32),
          pltpu.VMEM((2, AGRP, 128, W), jnp.float32),
          pltpu.VMEM((128, W), jnp.float32),
          pltpu.SemaphoreType.DMA,
          pltpu.SemaphoreType.DMA,
          pltpu.SemaphoreType.DMA,
      ],
      compiler_params=pltpu.CompilerParams(use_tc_tiling_on_sc=False),
  )
  def agg_kernel(zer_h, f0, f1, f2, s0, s1, s2, d0, d1, d2,
                 o0, o1, o2, acc, idxs, idxd, rows, zvm, gsem, ssem0, ssem1):
    feats = (f0, f1, f2)
    srcs = (s0, s1, s2)
    dsts = (d0, d1, d2)
    outs = (o0, o1, o2)
    sid = lax.axis_index("s")
    pltpu.sync_copy(zer_h, zvm)

    half = CHUNKS // 2
    ssems = (ssem0, ssem1)

    def task(feat, src, dst, out):
      def percol(c, carry):
        zd = [
            pltpu.async_copy(zvm,
                             acc.at[pl.ds(sid * STRIPE + i * 128, 128)],
                             ssem0)
            for i in range(STRIPE // 128)
        ]
        for d in zd:
          d.wait()
        plsc.subcore_barrier()

        for h in range(2):
          pltpu.sync_copy(src.at[sid, pl.ds(h * half, half)], idxs)
          pltpu.sync_copy(dst.at[sid, pl.ds(h * half, half)], idxd)

          def xform(j, xcarry):
            for k in range(8):
              sl = (j, pl.ds(k * 16, 16))
              idxs[sl] = idxs[sl] * 8 + c
            return xcarry
          lax.fori_loop(0, half, xform, 0)

          def body(g2, bcarry):
            for par in range(2):
              base = (g2 * 2 + par) * AGRP

              @pl.when(g2 > 0)
              def _():
                for k in range(AGRP):
                  pltpu.make_async_copy(rows.at[par, k],
                                        acc.at[idxd.at[k]],
                                        ssems[par]).wait()
              gd = [
                  pltpu.async_copy(feat.at[idxs.at[base + k]],
                                   rows.at[par, k], gsem)
                  for k in range(AGRP)
              ]
              for d in gd:
                d.wait()
              for k in range(AGRP):
                pltpu.async_copy(rows.at[par, k], acc.at[idxd.at[base + k]],
                                 ssems[par], add=True)
            return bcarry
          lax.fori_loop(0, half // (2 * AGRP), body, 0)
          for par in range(2):
            for k in range(AGRP):
              pltpu.make_async_copy(rows.at[par, k], acc.at[idxd.at[k]],
                                    ssems[par]).wait()

        plsc.subcore_barrier()
        pltpu.sync_copy(acc.at[pl.ds(sid * STRIPE, STRIPE)],
                        out.at[pl.ds(sid * STRIPE, STRIPE),
                               pl.ds(c * W, W)])
        plsc.subcore_barrier()
        return carry
      lax.fori_loop(0, ncols // W, percol, 0)

    for r in range(3):
      task(feats[r], srcs[r], dsts[r], outs[r])

  return agg_kernel


_agg_l1 = _make_agg(128)
_agg_l2 = _make_agg(48)


def _scale_body(d0, d1, d2, x_ref, f0, f1, f2):
  xv = x_ref[...]
  for dref, fref in ((d0, f0), (d1, f1), (d2, f2)):
    s = lax.rsqrt(jnp.maximum(dref[...], 1.0))
    fref[...] = xv * s


def _l1_body(i0, i1, i2, b10, b11, b12, w10, w11, w12, a0, a1, a2, h_ref):
  acc = jnp.zeros((TROW, 128), jnp.float32)
  for iref, wref, aref in ((i0, w10, a0), (i1, w11, a1), (i2, w12, a2)):
    s = lax.rsqrt(jnp.maximum(iref[...], 1.0))
    acc += jnp.dot(aref[...] * s, wref[...],
                   preferred_element_type=jnp.float32)
  bbar = (b10[...] + b11[...] + b12[...]) * (1.0 / 3.0)
  h_ref[...] = jnp.maximum(acc * (1.0 / 3.0) + bbar, 0.0)


def _l2pre_body(o0, o1, o2, wc_ref, w20, w21, w22, h_ref, q0, q1, q2):
  hv = h_ref[...]
  wcp = jnp.concatenate(
      [wc_ref[...], jnp.zeros((128, 88), jnp.float32)], axis=1)
  for oref, wref, qref in ((o0, w20, q0), (o1, w21, q1), (o2, w22, q2)):
    m = jnp.dot(wref[...], wcp, preferred_element_type=jnp.float32)
    s = lax.rsqrt(jnp.maximum(oref[...], 1.0))
    qref[...] = jnp.dot(hv * s, m, preferred_element_type=jnp.float32)


def _final_body(i0, i1, i2, b20, b21, b22, wc_ref, bc_ref, a0, a1, a2, out):
  acc = jnp.zeros((TROW, 48), jnp.float32)
  for iref, aref in ((i0, a0), (i1, a1), (i2, a2)):
    s = lax.rsqrt(jnp.maximum(iref[...], 1.0))
    acc += aref[...][:, :48] * s
  bb = jnp.dot((b20[...] + b21[...] + b22[...]) * (1.0 / 3.0), wc_ref[...],
               preferred_element_type=jnp.float32)
  out[...] = acc[:, :40] * (1.0 / 3.0) + bb + bc_ref[...]


def _row_spec(w):
  return pl.BlockSpec((TROW, w), lambda i: (i, 0))


def _full_spec(a, b):
  return pl.BlockSpec((a, b), lambda i: (0, 0))


def kernel(x, edge_index_r0, edge_index_r1, edge_index_r2,
           W1_0, b1_0, W1_1, b1_1, W1_2, b1_2,
           W2_0, b2_0, W2_1, b2_1, W2_2, b2_2,
           Wc, bc):
  xp = jnp.pad(x, ((0, NP - N), (0, 0)))

  pad = DUMMY + (jnp.arange(EP - E, dtype=jnp.int32) % (NP - N - 8))

  def prep(ei):
    s = jnp.concatenate([ei[0].astype(jnp.int32), pad]).reshape(16, CHUNKS, 128)
    d = jnp.concatenate([ei[1].astype(jnp.int32), pad]).reshape(16, CHUNKS, 128)
    return s, d

  s0, d0 = prep(edge_index_r0)
  s1, d1 = prep(edge_index_r1)
  s2, d2 = prep(edge_index_r2)

  ones128 = jnp.ones((128,), jnp.float32)
  zer1d = jnp.zeros((STRIPE,), jnp.float32)
  zer2d = jnp.zeros((128, W), jnp.float32)

  od0, od1, od2, id0, id1, id2 = _deg_kernel(
      ones128, zer1d, s0, s1, s2, d0, d1, d2)
  od = [d.reshape(NP, 1) for d in (od0, od1, od2)]
  idg = [d.reshape(NP, 1) for d in (id0, id1, id2)]

  feats = pl.pallas_call(
      _scale_body,
      grid=(NP // TROW,),
      in_specs=[_row_spec(1)] * 3 + [_row_spec(128)],
      out_specs=[_row_spec(128)] * 3,
      out_shape=[jax.ShapeDtypeStruct((NP, 128), jnp.float32)] * 3,
  )(od[0], od[1], od[2], xp)

  fviews = [f.reshape(8 * NP, W) for f in feats]
  aggs1 = _agg_l1(zer2d, *fviews, s0, s1, s2, d0, d1, d2)

  b1 = [b.reshape(1, 128) for b in (b1_0, b1_1, b1_2)]
  h = pl.pallas_call(
      _l1_body,
      grid=(NP // TROW,),
      in_specs=([_row_spec(1)] * 3 + [_full_spec(1, 128)] * 3
                + [_full_spec(128, 128)] * 3 + [_row_spec(128)] * 3),
      out_specs=_row_spec(128),
      out_shape=jax.ShapeDtypeStruct((NP, 128), jnp.float32),
  )(idg[0], idg[1], idg[2], *b1, W1_0, W1_1, W1_2, *aggs1)

  qs = pl.pallas_call(
      _l2pre_body,
      grid=(NP // TROW,),
      in_specs=([_row_spec(1)] * 3 + [_full_spec(128, 40)]
                + [_full_spec(128, 128)] * 3 + [_row_spec(128)]),
      out_specs=[_row_spec(128)] * 3,
      out_shape=[jax.ShapeDtypeStruct((NP, 128), jnp.float32)] * 3,
  )(od[0], od[1], od[2], Wc, W2_0, W2_1, W2_2, h)

  qviews = [q.reshape(8 * NP, W) for q in qs]
  aggs2 = _agg_l2(zer2d, *qviews, s0, s1, s2, d0, d1, d2)

  b2 = [b.reshape(1, 128) for b in (b2_0, b2_1, b2_2)]
  logits = pl.pallas_call(
      _final_body,
      grid=(NP // TROW,),
      in_specs=([_row_spec(1)] * 3 + [_full_spec(1, 128)] * 3
                + [_full_spec(128, 40)] + [_full_spec(1, 40)]
                + [_row_spec(128)] * 3),
      out_specs=_row_spec(40),
      out_shape=jax.ShapeDtypeStruct((NP, 40), jnp.float32),
  )(idg[0], idg[1], idg[2], *b2, Wc, bc.reshape(1, 40), *aggs2)

  return logits[:N]
